# Initial kernel scaffold; baseline (speedup 1.0000x reference)
import jax
import jax.numpy as jnp
from jax.experimental import pallas as pl


def kernel(Q, K, V, Wq, bq, Wk, bk, Wv, bv, Wo, bo):
    raise NotImplementedError("write your pallas kernel here")


# speedup vs baseline: 4.6248x; 4.6248x over previous
import math

import jax
import jax.numpy as jnp
from jax.experimental import pallas as pl

N_HEADS = 16
D_MODEL = 1024
DK = D_MODEL // N_HEADS


def _to_heads(x):
    bm = x.shape[0]
    return x.reshape(bm, N_HEADS, DK).transpose(1, 0, 2)


def _proj_kernel(x_q, x_k, x_v, wq, wk, wv, bq, bk, bv, oq, ok, ov):
    dn = (((1,), (1,)), ((), ()))
    oq[:] = _to_heads(jax.lax.dot_general(
        x_q[:], wq[:], dn, preferred_element_type=jnp.float32) + bq[:])
    ok[:] = _to_heads(jax.lax.dot_general(
        x_k[:], wk[:], dn, preferred_element_type=jnp.float32) + bk[:])
    ov[:] = _to_heads(jax.lax.dot_general(
        x_v[:], wv[:], dn, preferred_element_type=jnp.float32) + bv[:])


def _attn_kernel(q_ref, k_ref, v_ref, o_ref):
    q = q_ref[0]
    k = k_ref[0]
    s = jax.lax.dot_general(q, k, (((1,), (1,)), ((), ())),
                            preferred_element_type=jnp.float32)
    s = s * (1.0 / math.sqrt(DK))
    m = jnp.max(s, axis=-1, keepdims=True)
    p = jnp.exp(s - m)
    l = jnp.sum(p, axis=-1, keepdims=True)
    ctx = jnp.dot(p, v_ref[0], preferred_element_type=jnp.float32)
    o_ref[0] = ctx / l


def _oproj_kernel(x_ref, wo_ref, bo_ref, o_ref):
    bm = x_ref.shape[1]
    x = x_ref[:].transpose(1, 0, 2).reshape(bm, D_MODEL)
    dn = (((1,), (1,)), ((), ()))
    o_ref[:] = jax.lax.dot_general(x, wo_ref[:], dn,
                                   preferred_element_type=jnp.float32) + bo_ref[:]


def kernel(Q, K, V, Wq, bq, Wk, bk, Wv, bv, Wo, bo):
    B, L, D = Q.shape
    H, dk = N_HEADS, DK
    x_q = Q.reshape(L, D)
    x_k = K.reshape(L, D)
    x_v = V.reshape(L, D)
    bq2 = bq.reshape(1, D)
    bk2 = bk.reshape(1, D)
    bv2 = bv.reshape(1, D)
    bo2 = bo.reshape(1, D)

    BM = 512
    n_rb = L // BM

    w_spec = pl.BlockSpec((D, D), lambda i: (0, 0))
    b_spec = pl.BlockSpec((1, D), lambda i: (0, 0))
    row_spec = pl.BlockSpec((BM, D), lambda i: (i, 0))
    heads_spec = pl.BlockSpec((H, BM, dk), lambda i: (0, i, 0))

    qp, kp, vp = pl.pallas_call(
        _proj_kernel,
        grid=(n_rb,),
        in_specs=[row_spec, row_spec, row_spec,
                  w_spec, w_spec, w_spec,
                  b_spec, b_spec, b_spec],
        out_specs=[heads_spec, heads_spec, heads_spec],
        out_shape=[jax.ShapeDtypeStruct((H, L, dk), jnp.float32)] * 3,
    )(x_q, x_k, x_v, Wq, Wk, Wv, bq2, bk2, bv2)

    BQ = 512
    n_qb = L // BQ
    ctx = pl.pallas_call(
        _attn_kernel,
        grid=(H, n_qb),
        in_specs=[
            pl.BlockSpec((1, BQ, dk), lambda h, qb: (h, qb, 0)),
            pl.BlockSpec((1, L, dk), lambda h, qb: (h, 0, 0)),
            pl.BlockSpec((1, L, dk), lambda h, qb: (h, 0, 0)),
        ],
        out_specs=pl.BlockSpec((1, BQ, dk), lambda h, qb: (h, qb, 0)),
        out_shape=jax.ShapeDtypeStruct((H, L, dk), jnp.float32),
    )(qp, kp, vp)

    out = pl.pallas_call(
        _oproj_kernel,
        grid=(n_rb,),
        in_specs=[heads_spec, w_spec, b_spec],
        out_specs=row_spec,
        out_shape=jax.ShapeDtypeStruct((L, D), jnp.float32),
    )(ctx, Wo, bo2)

    return out.reshape(B, L, D)
